# Initial kernel scaffold; baseline (speedup 1.0000x reference)
import jax
import jax.numpy as jnp
from jax.experimental import pallas as pl


def kernel(x, adj, W, b):
    raise NotImplementedError("write your pallas kernel here")


# speedup vs baseline: 1.0077x; 1.0077x over previous
import jax
import jax.numpy as jnp
from jax.experimental import pallas as pl
from jax.experimental.pallas import tpu as pltpu

N = 10000
D = 128
NCLASS = 2
BLK = 400


NBLK = N // BLK


def _lightgcn_body(x_ref, adj_ref, w_ref, b_ref, out_ref, z_ref, t1_ref):
    i = pl.program_id(0)
    phase = i // NBLK
    k = i % NBLK

    @pl.when((phase == 0) & (k == 0))
    def _():
        z_ref[...] = jnp.dot(
            x_ref[...], w_ref[...], preferred_element_type=jnp.float32
        )

    @pl.when(phase == 0)
    def _():
        t1_ref[pl.ds(k * BLK, BLK), :] = jnp.dot(
            adj_ref[...], z_ref[...], preferred_element_type=jnp.float32
        )

    @pl.when(phase == 1)
    def _():
        logits = (
            jnp.dot(adj_ref[...], t1_ref[...], preferred_element_type=jnp.float32)
            + b_ref[...]
        )
        m = jnp.max(logits, axis=1, keepdims=True)
        lse = m + jnp.log(jnp.sum(jnp.exp(logits - m), axis=1, keepdims=True))
        out_ref[...] = logits - lse


def kernel(x, adj, W, b):
    b2d = b.reshape(1, NCLASS)
    grid = (2 * NBLK,)
    return pl.pallas_call(
        _lightgcn_body,
        grid=grid,
        in_specs=[
            pl.BlockSpec((N, D), lambda i: (0, 0)),
            pl.BlockSpec((BLK, N), lambda i: (i % NBLK, 0)),
            pl.BlockSpec((D, NCLASS), lambda i: (0, 0)),
            pl.BlockSpec((1, NCLASS), lambda i: (0, 0)),
        ],
        out_specs=pl.BlockSpec(
            (BLK, NCLASS), lambda i: (jnp.maximum(i - NBLK, 0), 0)
        ),
        out_shape=jax.ShapeDtypeStruct((N, NCLASS), jnp.float32),
        scratch_shapes=[
            pltpu.VMEM((N, NCLASS), jnp.float32),
            pltpu.VMEM((N, NCLASS), jnp.float32),
        ],
    )(x, adj, W, b2d)
